# baseline (speedup 1.0000x reference)
import functools

import jax
import jax.numpy as jnp
from jax import lax
from jax.experimental import pallas as pl
from jax.experimental.pallas import tpu as pltpu
from jax.experimental.pallas import tpu_sc as plsc

CHUNK = 128
GROUP = 2
N_SB = 3


@functools.lru_cache(maxsize=None)
def _make_gather(V, D, B):
    info = plsc.get_sparse_core_info()
    NC, NS = info.num_cores, info.num_subcores
    NW = NC * NS
    assert B % (NW * CHUNK * GROUP) == 0
    n_chunks = B // (NW * CHUNK)
    n_sb = n_chunks // GROUP
    n_main = (n_sb - 4) // N_SB
    assert n_sb - 4 == n_main * N_SB and n_sb >= 6
    mesh = plsc.VectorSubcoreMesh(core_axis_name="c", subcore_axis_name="s")

    @functools.partial(
        pl.kernel,
        mesh=mesh,
        out_type=jax.ShapeDtypeStruct((B, D), jnp.float32),
        scratch_types=[
            pltpu.VMEM((n_chunks, CHUNK), jnp.int32),
        ]
        + [pltpu.VMEM((GROUP * CHUNK, D), jnp.float32)] * N_SB
        + [pltpu.SemaphoreType.DMA] * (2 * N_SB),
    )
    def grab(x_hbm, w_hbm, out_hbm, idx_v, *rest):
        sbs = rest[:N_SB]
        gsems = rest[N_SB : 2 * N_SB]
        osems = rest[2 * N_SB :]
        wid = lax.axis_index("s") * NC + lax.axis_index("c")
        pltpu.sync_copy(x_hbm.at[wid], idx_v)
        out_base = wid * (n_chunks * CHUNK)

        def start_gathers(k, slot):
            for h in range(GROUP):
                pltpu.async_copy(
                    w_hbm.at[idx_v.at[k * GROUP + h]],
                    sbs[slot].at[pl.ds(h * CHUNK, CHUNK)],
                    gsems[slot],
                )

        def wait_gathers(k, slot):
            for h in range(GROUP):
                pltpu.make_async_copy(
                    w_hbm.at[idx_v.at[k * GROUP + h]],
                    sbs[slot].at[pl.ds(h * CHUNK, CHUNK)],
                    gsems[slot],
                ).wait()

        def out_slice(k):
            return out_hbm.at[pl.ds(out_base + k * (GROUP * CHUNK), GROUP * CHUNK)]

        def start_write(k, slot):
            pltpu.async_copy(sbs[slot], out_slice(k), osems[slot])

        def wait_write(k, slot):
            pltpu.make_async_copy(sbs[slot], out_slice(k), osems[slot]).wait()

        def retire(k, slot):
            wait_gathers(k, slot)
            start_write(k, slot)

        start_gathers(0, 0)
        start_gathers(1, 1)
        retire(0, 0)
        start_gathers(2, 2)
        retire(1, 1)

        def body(i, carry):
            for off in range(N_SB):
                k = 2 + N_SB * i + off
                slot = (2 + off) % N_SB
                nxt = (slot + 1) % N_SB
                wait_write(k - 2, nxt)
                start_gathers(k + 1, nxt)
                retire(k, slot)
            return carry

        lax.fori_loop(0, n_main, body, 0)

        for k in range(n_sb - 2, n_sb):
            slot = k % N_SB
            if k + 1 < n_sb:
                nxt = (k + 1) % N_SB
                wait_write(k - 2, nxt)
                start_gathers(k + 1, nxt)
            retire(k, slot)
        for k in range(n_sb - N_SB, n_sb):
            wait_write(k, k % N_SB)

    return grab


def kernel(x, weight):
    B, S = x.shape
    V, D = weight.shape
    total = B * S
    info = plsc.get_sparse_core_info()
    NW = info.num_cores * info.num_subcores
    n_chunks = total // (NW * CHUNK)
    xf = x.astype(jnp.int32).reshape(NW, n_chunks, CHUNK)
    out = _make_gather(V, D, total)(xf, weight)
    return out.reshape(B, S, D)


# speedup vs baseline: 1.0274x; 1.0274x over previous
import functools

import jax
import jax.numpy as jnp
from jax import lax
from jax.experimental import pallas as pl
from jax.experimental.pallas import tpu as pltpu
from jax.experimental.pallas import tpu_sc as plsc

CHUNK = 128
N_BUF = 7
AHEAD = 6


@functools.lru_cache(maxsize=None)
def _make_gather(V, D, B):
    info = plsc.get_sparse_core_info()
    NC, NS = info.num_cores, info.num_subcores
    NW = NC * NS
    assert B % (NW * CHUNK) == 0
    n_chunks = B // (NW * CHUNK)
    assert 0 < AHEAD < N_BUF
    H = AHEAD + (n_chunks - AHEAD - N_BUF) % N_BUF
    n_main = (n_chunks - H - N_BUF) // N_BUF
    assert n_main >= 0 and H + AHEAD <= n_chunks and H >= N_BUF - AHEAD
    mesh = plsc.VectorSubcoreMesh(core_axis_name="c", subcore_axis_name="s")

    @functools.partial(
        pl.kernel,
        mesh=mesh,
        out_type=jax.ShapeDtypeStruct((B, D), jnp.float32),
        scratch_types=[
            pltpu.VMEM((n_chunks, CHUNK), jnp.int32),
        ]
        + [pltpu.VMEM((CHUNK, D), jnp.float32)] * N_BUF
        + [pltpu.SemaphoreType.DMA] * (2 * N_BUF),
    )
    def grab(x_hbm, w_hbm, out_hbm, idx_v, *rest):
        bufs = rest[:N_BUF]
        gsems = rest[N_BUF : 2 * N_BUF]
        osems = rest[2 * N_BUF :]
        wid = lax.axis_index("s") * NC + lax.axis_index("c")
        pltpu.sync_copy(x_hbm.at[wid], idx_v)
        out_base = wid * (n_chunks * CHUNK)

        def start_gather(j, slot):
            pltpu.async_copy(w_hbm.at[idx_v.at[j]], bufs[slot], gsems[slot])

        def wait_gather(j, slot):
            pltpu.make_async_copy(
                w_hbm.at[idx_v.at[j]], bufs[slot], gsems[slot]
            ).wait()

        def out_slice(j):
            return out_hbm.at[pl.ds(out_base + j * CHUNK, CHUNK)]

        def start_out(j, slot):
            pltpu.async_copy(bufs[slot], out_slice(j), osems[slot])

        def wait_out(j, slot):
            pltpu.make_async_copy(bufs[slot], out_slice(j), osems[slot]).wait()

        def retire(j, slot):
            wait_gather(j, slot)
            start_out(j, slot)

        for j in range(AHEAD):
            start_gather(j, j % N_BUF)
        for j in range(H):
            jg = j + AHEAD
            csl = jg % N_BUF
            if jg - N_BUF >= 0:
                wait_out(jg - N_BUF, csl)
            start_gather(jg, csl)
            retire(j, j % N_BUF)

        def body(i, carry):
            for b_off in range(N_BUF):
                j = H + N_BUF * i + b_off
                slot = (H + b_off) % N_BUF
                csl = (slot + AHEAD) % N_BUF
                wait_out(j + AHEAD - N_BUF, csl)
                start_gather(j + AHEAD, csl)
                retire(j, slot)
            return carry

        lax.fori_loop(0, n_main, body, 0)

        for j in range(n_chunks - N_BUF, n_chunks):
            jg = j + AHEAD
            if jg < n_chunks:
                csl = jg % N_BUF
                wait_out(jg - N_BUF, csl)
                start_gather(jg, csl)
            retire(j, j % N_BUF)
        for j in range(n_chunks - N_BUF, n_chunks):
            wait_out(j, j % N_BUF)

    return grab


def kernel(x, weight):
    B, S = x.shape
    V, D = weight.shape
    total = B * S
    info = plsc.get_sparse_core_info()
    NW = info.num_cores * info.num_subcores
    n_chunks = total // (NW * CHUNK)
    xf = x.astype(jnp.int32).reshape(NW, n_chunks, CHUNK)
    out = _make_gather(V, D, total)(xf, weight)
    return out.reshape(B, S, D)
